# baseline (speedup 1.0000x reference)
import jax
import jax.numpy as jnp
from jax import lax
from jax.experimental import pallas as pl

N = 4096
IN_FEAT = 512
HID_FEAT = 256
OUT_FEAT = 64
CT = 20

RB_EMBED = 512
RB_ADJ = 256
RB_DEC = 256


def _embed_kernel(feat_ref, w1_ref, w2_ref, g_ref):
    h = jnp.dot(feat_ref[...], w1_ref[...], preferred_element_type=jnp.float32)
    g_ref[...] = jnp.dot(h, w2_ref[...], preferred_element_type=jnp.float32)


def _adjmm_kernel(adj_ref, v_ref, o_ref):
    o_ref[...] = jnp.dot(adj_ref[...], v_ref[...],
                         preferred_element_type=jnp.float32)


def _decode_kernel(z_blk_ref, z_all_ref, wp_ref, bp_ref, wr_ref, br_ref,
                   arec_ref, pred_ref, rec_ref):
    zb = z_blk_ref[...]
    prod = lax.dot_general(zb, z_all_ref[...],
                           (((1,), (1,)), ((), ())),
                           preferred_element_type=jnp.float32)
    arec_ref[...] = jax.nn.sigmoid(prod)
    logits = jnp.dot(zb, wp_ref[...],
                     preferred_element_type=jnp.float32) + bp_ref[...]
    pred_ref[...] = jax.nn.softmax(logits, axis=-1)
    rec_ref[...] = jnp.dot(zb, wr_ref[...],
                           preferred_element_type=jnp.float32) + br_ref[...]


def _embed(feat, W1, W2):
    return pl.pallas_call(
        _embed_kernel,
        grid=(N // RB_EMBED,),
        in_specs=[
            pl.BlockSpec((RB_EMBED, IN_FEAT), lambda i: (i, 0)),
            pl.BlockSpec((IN_FEAT, HID_FEAT), lambda i: (0, 0)),
            pl.BlockSpec((HID_FEAT, OUT_FEAT), lambda i: (0, 0)),
        ],
        out_specs=pl.BlockSpec((RB_EMBED, OUT_FEAT), lambda i: (i, 0)),
        out_shape=jax.ShapeDtypeStruct((N, OUT_FEAT), jnp.float32),
    )(feat, W1, W2)


def _adjmm(adj, v):
    return pl.pallas_call(
        _adjmm_kernel,
        grid=(N // RB_ADJ,),
        in_specs=[
            pl.BlockSpec((RB_ADJ, N), lambda i: (i, 0)),
            pl.BlockSpec((N, OUT_FEAT), lambda i: (0, 0)),
        ],
        out_specs=pl.BlockSpec((RB_ADJ, OUT_FEAT), lambda i: (i, 0)),
        out_shape=jax.ShapeDtypeStruct((N, OUT_FEAT), jnp.float32),
    )(adj, v)


def _decode(z, Wp, bp2, Wr, br2):
    return pl.pallas_call(
        _decode_kernel,
        grid=(N // RB_DEC,),
        in_specs=[
            pl.BlockSpec((RB_DEC, OUT_FEAT), lambda i: (i, 0)),
            pl.BlockSpec((N, OUT_FEAT), lambda i: (0, 0)),
            pl.BlockSpec((OUT_FEAT, CT), lambda i: (0, 0)),
            pl.BlockSpec((1, CT), lambda i: (0, 0)),
            pl.BlockSpec((OUT_FEAT, IN_FEAT), lambda i: (0, 0)),
            pl.BlockSpec((1, IN_FEAT), lambda i: (0, 0)),
        ],
        out_specs=[
            pl.BlockSpec((RB_DEC, N), lambda i: (i, 0)),
            pl.BlockSpec((RB_DEC, CT), lambda i: (i, 0)),
            pl.BlockSpec((RB_DEC, IN_FEAT), lambda i: (i, 0)),
        ],
        out_shape=[
            jax.ShapeDtypeStruct((N, N), jnp.float32),
            jax.ShapeDtypeStruct((N, CT), jnp.float32),
            jax.ShapeDtypeStruct((N, IN_FEAT), jnp.float32),
        ],
    )(z, z, Wp, bp2, Wr, br2)


def kernel(features, features_sc, adj_spatial, adj_feature, W1, W2, Wp, bp, Wr, br):
    bp2 = bp.reshape(1, CT)
    br2 = br.reshape(1, IN_FEAT)

    def encoder_and_heads(feat, adj):
        g = _embed(feat, W1, W2)
        q = _adjmm(adj, g)
        z = _adjmm(adj, q)
        a_rec, pred, rec = _decode(z, Wp, bp2, Wr, br2)
        return z, a_rec, pred, rec

    latent_spatial, spatial_graph_rec, pred_st, spatial_rec = encoder_and_heads(
        features, adj_spatial)
    latent_feature, feature_graph_rec, pred_sc, feature_rec = encoder_and_heads(
        features_sc, adj_feature)

    return (latent_spatial, latent_feature, spatial_rec, feature_rec,
            spatial_graph_rec, feature_graph_rec, pred_st, pred_sc)


# speedup vs baseline: 1.2107x; 1.2107x over previous
import jax
import jax.numpy as jnp
from jax import lax
from jax.experimental import pallas as pl

N = 4096
IN_FEAT = 512
HID_FEAT = 256
OUT_FEAT = 64
CT = 20

RB = 512


def _embed_kernel(fs_ref, ff_ref, w1_ref, w2_ref, gs_ref, gf_ref):
    w1 = w1_ref[...]
    w2 = w2_ref[...]
    hs = jnp.dot(fs_ref[...], w1, preferred_element_type=jnp.float32)
    gs_ref[...] = jnp.dot(hs, w2, preferred_element_type=jnp.float32)
    hf = jnp.dot(ff_ref[...], w1, preferred_element_type=jnp.float32)
    gf_ref[...] = jnp.dot(hf, w2, preferred_element_type=jnp.float32)


def _adjmm_kernel(as_ref, af_ref, vs_ref, vf_ref, os_ref, of_ref):
    os_ref[...] = jnp.dot(as_ref[...], vs_ref[...],
                          preferred_element_type=jnp.float32)
    of_ref[...] = jnp.dot(af_ref[...], vf_ref[...],
                          preferred_element_type=jnp.float32)


def _decode_one(zb, z_all, wp, bp, wr, br, arec_ref, pred_ref, rec_ref):
    prod = lax.dot_general(zb, z_all, (((1,), (1,)), ((), ())),
                           preferred_element_type=jnp.float32)
    arec_ref[...] = jax.nn.sigmoid(prod)
    logits = jnp.dot(zb, wp, preferred_element_type=jnp.float32) + bp
    pred_ref[...] = jax.nn.softmax(logits, axis=-1)
    rec_ref[...] = jnp.dot(zb, wr, preferred_element_type=jnp.float32) + br


def _decode_kernel(zbs_ref, zbf_ref, zs_ref, zf_ref, wp_ref, bp_ref, wr_ref,
                   br_ref, arecs_ref, preds_ref, recs_ref,
                   arecf_ref, predf_ref, recf_ref):
    wp = wp_ref[...]
    bp = bp_ref[...]
    wr = wr_ref[...]
    br = br_ref[...]
    _decode_one(zbs_ref[...], zs_ref[...], wp, bp, wr, br,
                arecs_ref, preds_ref, recs_ref)
    _decode_one(zbf_ref[...], zf_ref[...], wp, bp, wr, br,
                arecf_ref, predf_ref, recf_ref)


def _embed(feat_s, feat_f, W1, W2):
    return pl.pallas_call(
        _embed_kernel,
        grid=(N // RB,),
        in_specs=[
            pl.BlockSpec((RB, IN_FEAT), lambda i: (i, 0)),
            pl.BlockSpec((RB, IN_FEAT), lambda i: (i, 0)),
            pl.BlockSpec((IN_FEAT, HID_FEAT), lambda i: (0, 0)),
            pl.BlockSpec((HID_FEAT, OUT_FEAT), lambda i: (0, 0)),
        ],
        out_specs=[
            pl.BlockSpec((RB, OUT_FEAT), lambda i: (i, 0)),
            pl.BlockSpec((RB, OUT_FEAT), lambda i: (i, 0)),
        ],
        out_shape=[
            jax.ShapeDtypeStruct((N, OUT_FEAT), jnp.float32),
            jax.ShapeDtypeStruct((N, OUT_FEAT), jnp.float32),
        ],
    )(feat_s, feat_f, W1, W2)


def _adjmm(adj_s, adj_f, v_s, v_f):
    return pl.pallas_call(
        _adjmm_kernel,
        grid=(N // RB,),
        in_specs=[
            pl.BlockSpec((RB, N), lambda i: (i, 0)),
            pl.BlockSpec((RB, N), lambda i: (i, 0)),
            pl.BlockSpec((N, OUT_FEAT), lambda i: (0, 0)),
            pl.BlockSpec((N, OUT_FEAT), lambda i: (0, 0)),
        ],
        out_specs=[
            pl.BlockSpec((RB, OUT_FEAT), lambda i: (i, 0)),
            pl.BlockSpec((RB, OUT_FEAT), lambda i: (i, 0)),
        ],
        out_shape=[
            jax.ShapeDtypeStruct((N, OUT_FEAT), jnp.float32),
            jax.ShapeDtypeStruct((N, OUT_FEAT), jnp.float32),
        ],
    )(adj_s, adj_f, v_s, v_f)


def _decode(z_s, z_f, Wp, bp2, Wr, br2):
    return pl.pallas_call(
        _decode_kernel,
        grid=(N // RB,),
        in_specs=[
            pl.BlockSpec((RB, OUT_FEAT), lambda i: (i, 0)),
            pl.BlockSpec((RB, OUT_FEAT), lambda i: (i, 0)),
            pl.BlockSpec((N, OUT_FEAT), lambda i: (0, 0)),
            pl.BlockSpec((N, OUT_FEAT), lambda i: (0, 0)),
            pl.BlockSpec((OUT_FEAT, CT), lambda i: (0, 0)),
            pl.BlockSpec((1, CT), lambda i: (0, 0)),
            pl.BlockSpec((OUT_FEAT, IN_FEAT), lambda i: (0, 0)),
            pl.BlockSpec((1, IN_FEAT), lambda i: (0, 0)),
        ],
        out_specs=[
            pl.BlockSpec((RB, N), lambda i: (i, 0)),
            pl.BlockSpec((RB, CT), lambda i: (i, 0)),
            pl.BlockSpec((RB, IN_FEAT), lambda i: (i, 0)),
            pl.BlockSpec((RB, N), lambda i: (i, 0)),
            pl.BlockSpec((RB, CT), lambda i: (i, 0)),
            pl.BlockSpec((RB, IN_FEAT), lambda i: (i, 0)),
        ],
        out_shape=[
            jax.ShapeDtypeStruct((N, N), jnp.float32),
            jax.ShapeDtypeStruct((N, CT), jnp.float32),
            jax.ShapeDtypeStruct((N, IN_FEAT), jnp.float32),
            jax.ShapeDtypeStruct((N, N), jnp.float32),
            jax.ShapeDtypeStruct((N, CT), jnp.float32),
            jax.ShapeDtypeStruct((N, IN_FEAT), jnp.float32),
        ],
    )(z_s, z_f, z_s, z_f, Wp, bp2, Wr, br2)


def kernel(features, features_sc, adj_spatial, adj_feature, W1, W2, Wp, bp, Wr, br):
    bp2 = bp.reshape(1, CT)
    br2 = br.reshape(1, IN_FEAT)

    g_s, g_f = _embed(features, features_sc, W1, W2)
    q_s, q_f = _adjmm(adj_spatial, adj_feature, g_s, g_f)
    z_s, z_f = _adjmm(adj_spatial, adj_feature, q_s, q_f)
    arec_s, pred_s, rec_s, arec_f, pred_f, rec_f = _decode(
        z_s, z_f, Wp, bp2, Wr, br2)

    return (z_s, z_f, rec_s, rec_f, arec_s, arec_f, pred_s, pred_f)
